# baseline (speedup 1.0000x reference)
import functools

import jax
import jax.numpy as jnp
from jax import lax
from jax.experimental import pallas as pl
from jax.experimental.pallas import tpu as pltpu
from jax.experimental.pallas import tpu_sc as plsc

NUM_EMB = 1000000
DIM = 32
BATCH = 16384
FIELDS = 26
NW = 32
NS = BATCH // NW
SCH = 256
NCK = NS // SCH
WIDE = 4 * DIM

_mesh = plsc.VectorSubcoreMesh(core_axis_name="c", subcore_axis_name="s")


@functools.partial(
    pl.kernel,
    mesh=_mesh,
    out_type=jax.ShapeDtypeStruct((FIELDS, DIM, BATCH), jnp.float32),
    scratch_types=[
        pltpu.VMEM((1, NS), jnp.int32),
        pltpu.VMEM((SCH,), jnp.int32),
        pltpu.VMEM((SCH,), jnp.int32),
        pltpu.VMEM((SCH, WIDE), jnp.float32),
        pltpu.VMEM((1, DIM, SCH), jnp.float32),
        pltpu.SemaphoreType.DMA,
    ],
    compiler_params=pltpu.CompilerParams(
        use_tc_tiling_on_sc=True, needs_layout_passes=False),
)
def _fused(xt_hbm, tw_hbm, out_hbm, idxv, cidx, subv, gbuf, slab, sem):
    wid = lax.axis_index("s") * 2 + lax.axis_index("c")
    s0 = wid * NS
    lane = lax.iota(jnp.int32, 16)

    def f_body(f, carry):
        pltpu.sync_copy(xt_hbm.at[pl.ds(f, 1), pl.ds(s0, NS)], idxv)

        def c_body(c, carry):
            def prep(g, carry):
                v = idxv[0, pl.ds(c * SCH + g * 16, 16)]
                cidx[pl.ds(g * 16, 16)] = lax.shift_right_logical(v, 2)
                subv[pl.ds(g * 16, 16)] = (v & 3) * DIM
                return carry

            lax.fori_loop(0, SCH // 16, prep, 0)
            pltpu.async_copy(tw_hbm.at[cidx], gbuf, sem).wait()

            def g_body(g, carry):
                sv = subv[pl.ds(g * 16, 16)]
                rowv = lane + g * 16
                for d in range(DIM):
                    vals = plsc.load_gather(gbuf, [rowv, sv + d])
                    slab[0, d, pl.ds(g * 16, 16)] = vals
                return carry

            lax.fori_loop(0, SCH // 16, g_body, 0)
            pltpu.sync_copy(
                slab, out_hbm.at[pl.ds(f, 1), :, pl.ds(s0 + c * SCH, SCH)])
            return carry

        lax.fori_loop(0, NCK, c_body, 0)
        return carry

    lax.fori_loop(0, FIELDS, f_body, 0)


def kernel(x, table):
    table_wide = table.reshape(NUM_EMB // 4, WIDE)
    out = _fused(x.T, table_wide)
    return out.transpose(2, 0, 1)


# speedup vs baseline: 1.3123x; 1.3123x over previous
import functools

import jax
import jax.numpy as jnp
from jax import lax
from jax.experimental import pallas as pl
from jax.experimental.pallas import tpu as pltpu
from jax.experimental.pallas import tpu_sc as plsc

NUM_EMB = 1000000
DIM = 32
BATCH = 16384
FIELDS = 26
B = BATCH * FIELDS
NW = 32
BPW = B // NW
NB = 4
CH = 832
NCH = BPW // CH

_mesh = plsc.VectorSubcoreMesh(core_axis_name="c", subcore_axis_name="s")


@functools.partial(
    pl.kernel,
    mesh=_mesh,
    out_type=jax.ShapeDtypeStruct((B, DIM), jnp.float32),
    scratch_types=[
        pltpu.VMEM((BPW,), jnp.int32),
        pltpu.VMEM((NB, CH, DIM), jnp.float32),
        pltpu.SemaphoreType.DMA((NB,)),
        pltpu.SemaphoreType.DMA((NB,)),
    ],
    compiler_params=pltpu.CompilerParams(use_tc_tiling_on_sc=False),
)
def _gather_kernel(idx_hbm, table_hbm, out_hbm, idx_v, bufs, gsem, ssem):
    wid = lax.axis_index("s") * 2 + lax.axis_index("c")
    base = wid * BPW
    pltpu.sync_copy(idx_hbm.at[pl.ds(base, BPW)], idx_v)

    gat = [None] * NCH
    scat = [None] * NCH

    def issue_gather(c):
        b = c % NB
        gat[c] = pltpu.async_copy(
            table_hbm.at[idx_v.at[pl.ds(c * CH, CH)]], bufs.at[b], gsem.at[b])

    issue_gather(0)
    for c in range(NCH):
        if c + 1 < NCH:
            if c + 1 >= NB:
                scat[c + 1 - NB].wait()
            issue_gather(c + 1)
        gat[c].wait()
        b = c % NB
        scat[c] = pltpu.async_copy(
            bufs.at[b], out_hbm.at[pl.ds(base + c * CH, CH)], ssem.at[b])
    for c in range(max(0, NCH - NB), NCH):
        scat[c].wait()


@functools.partial(
    pl.kernel,
    mesh=_mesh,
    out_type=jax.ShapeDtypeStruct((B,), jnp.int32),
    scratch_types=[pltpu.VMEM((BATCH,), jnp.int32)],
    compiler_params=pltpu.CompilerParams(use_tc_tiling_on_sc=True),
)
def _detile_idx(xt_hbm, flat_hbm, buf):
    wid = lax.axis_index("s") * 2 + lax.axis_index("c")

    @pl.when(wid < FIELDS)
    def _():
        pltpu.sync_copy(xt_hbm.at[wid], buf)
        pltpu.sync_copy(buf, flat_hbm.at[pl.ds(wid * BATCH, BATCH)])


def kernel(x, table):
    flat_idx = _detile_idx(x.T)
    table_wide = jax.lax.optimization_barrier(table.reshape(NUM_EMB // 4, 4 * DIM))
    out = _gather_kernel(flat_idx, table_wide.reshape(NUM_EMB, DIM))
    return out.reshape(FIELDS, BATCH, DIM).transpose(1, 0, 2)
